# baseline (speedup 1.0000x reference)
import functools

import jax
import jax.numpy as jnp
from jax import lax
from jax.experimental import pallas as pl
from jax.experimental.pallas import tpu as pltpu
from jax.experimental.pallas import tpu_sc as plsc

_N = 10000
_E = 320000
_H = 128
_G = 64

_NC = 2
_NS = 16
_NW = _NC * _NS
_CHUNK = 128
_NCHUNK = 80
_EPT = _NCHUNK * _CHUNK
_EPAD = _NW * _EPT
_NBUF = 2
_NPAD = 10112
_RPT = _NPAD // _NS

_BLK = 1000
_NBLK = _N // _BLK


def _sc_aggregate(z, srcp, dstp, zero):
  mesh = plsc.VectorSubcoreMesh(
      core_axis_name="c", subcore_axis_name="s", num_cores=_NC,
      num_subcores=_NS)

  @functools.partial(
      pl.kernel,
      mesh=mesh,
      out_type=jax.ShapeDtypeStruct((_NC, _NPAD, _H), jnp.float32),
      scratch_types=[
          pltpu.VMEM_SHARED((_NPAD, _H), jnp.float32),
          pltpu.VMEM((_NCHUNK, _CHUNK), jnp.int32),
      ] + [pltpu.VMEM((_CHUNK,), jnp.int32) for _ in range(_NBUF)]
        + [pltpu.VMEM((_CHUNK, _H), jnp.float32) for _ in range(_NBUF)]
        + [pltpu.SemaphoreType.DMA for _ in range(3 * _NBUF + 1)],
  )
  def agg_kernel(z_hbm, src_hbm, dst_hbm, zero_hbm, out_hbm,
                 acc, didx, *rest):
    sidx = rest[:_NBUF]
    rows = rest[_NBUF:2 * _NBUF]
    gsems = rest[2 * _NBUF:3 * _NBUF]
    ssems = rest[3 * _NBUF:4 * _NBUF]
    isems = rest[4 * _NBUF:5 * _NBUF]
    zsem = rest[5 * _NBUF]
    cid = lax.axis_index("c")
    sid = lax.axis_index("s")
    wid = sid * _NC + cid
    row0 = sid * _RPT
    pltpu.async_copy(zero_hbm.at[pl.ds(row0, _RPT)],
                     acc.at[pl.ds(row0, _RPT)], zsem)
    ebase = wid * _EPT
    pltpu.async_copy(dst_hbm.at[wid], didx, isems[0])
    for b in range(_NBUF):
      pltpu.async_copy(src_hbm.at[pl.ds(ebase + b * _CHUNK, _CHUNK)],
                       sidx[b], isems[b])
    pltpu.make_async_copy(dst_hbm.at[wid], didx, isems[0]).wait()
    for b in range(_NBUF):
      pltpu.make_async_copy(src_hbm.at[pl.ds(ebase + b * _CHUNK, _CHUNK)],
                            sidx[b], isems[b]).wait()
    pltpu.make_async_copy(zero_hbm.at[pl.ds(row0, _RPT)],
                          acc.at[pl.ds(row0, _RPT)], zsem).wait()
    plsc.subcore_barrier()

    for b in range(_NBUF):
      pltpu.async_copy(z_hbm.at[sidx[b]], rows[b], gsems[b])

    @pl.loop(0, _NCHUNK - _NBUF, step=_NBUF)
    def _(cc):
      for b in range(_NBUF):
        c = cc + b
        pltpu.make_async_copy(z_hbm.at[sidx[b]], rows[b], gsems[b]).wait()
        pltpu.async_copy(rows[b], acc.at[didx.at[c]], ssems[b], add=True)
        pltpu.async_copy(
            src_hbm.at[pl.ds(ebase + (c + _NBUF) * _CHUNK, _CHUNK)],
            sidx[b], isems[b])
      for b in range(_NBUF):
        c = cc + b
        pltpu.make_async_copy(rows[b], acc.at[didx.at[c]], ssems[b]).wait()
        pltpu.make_async_copy(
            src_hbm.at[pl.ds(ebase + (c + _NBUF) * _CHUNK, _CHUNK)],
            sidx[b], isems[b]).wait()
        pltpu.async_copy(z_hbm.at[sidx[b]], rows[b], gsems[b])

    for b in range(_NBUF):
      c = _NCHUNK - _NBUF + b
      pltpu.make_async_copy(z_hbm.at[sidx[b]], rows[b], gsems[b]).wait()
      pltpu.async_copy(rows[b], acc.at[didx.at[c]], ssems[b], add=True)
    for b in range(_NBUF):
      c = _NCHUNK - _NBUF + b
      pltpu.make_async_copy(rows[b], acc.at[didx.at[c]], ssems[b]).wait()

    plsc.subcore_barrier()
    pltpu.sync_copy(acc.at[pl.ds(row0, _RPT)],
                    out_hbm.at[cid, pl.ds(row0, _RPT)])

  return agg_kernel(z, srcp, dstp, zero)


def _mlp_body(z_ref, a0_ref, a1_ref, wa_ref, ba_ref, wb_ref, bb_ref,
              y_ref, s1_ref, s2_ref):
  i = pl.program_id(0)
  h = z_ref[...] + a0_ref[...] + a1_ref[...]
  u = jnp.maximum(
      jnp.dot(h, wa_ref[...], preferred_element_type=jnp.float32)
      + ba_ref[...], 0.0)
  y = jnp.maximum(
      jnp.dot(u, wb_ref[...], preferred_element_type=jnp.float32)
      + bb_ref[...], 0.0)
  y_ref[...] = y

  @pl.when(i == 0)
  def _():
    s1_ref[...] = jnp.zeros_like(s1_ref)
    s2_ref[...] = jnp.zeros_like(s2_ref)

  s1_ref[...] += jnp.sum(y, axis=0, keepdims=True)
  s2_ref[...] += jnp.sum(y * y, axis=0, keepdims=True)


def _mlp_call(z, a0, a1, wa, ba, wb, bb):
  full = pl.BlockSpec((1, _H), lambda i: (0, 0))
  wfull = pl.BlockSpec((_H, _H), lambda i: (0, 0))
  rows = pl.BlockSpec((_BLK, _H), lambda i: (i, 0))
  return pl.pallas_call(
      _mlp_body,
      grid=(_NBLK,),
      in_specs=[rows, rows, rows, wfull, full, wfull, full],
      out_specs=[rows, full, full],
      out_shape=[
          jax.ShapeDtypeStruct((_N, _H), jnp.float32),
          jax.ShapeDtypeStruct((1, _H), jnp.float32),
          jax.ShapeDtypeStruct((1, _H), jnp.float32),
      ],
  )(z, a0, a1, wa, ba, wb, bb)


def _bn_pool_body(y_ref, s1_ref, s2_ref, gm_ref, bt_ref, seg_ref,
                  z_ref, g_ref):
  i = pl.program_id(0)
  mu = s1_ref[...] * (1.0 / _N)
  var = s2_ref[...] * (1.0 / _N) - mu * mu
  a = gm_ref[...] / jnp.sqrt(var + 1e-5)
  b = bt_ref[...] - mu * a
  z = y_ref[...] * a + b
  z_ref[...] = z

  seg = seg_ref[0]
  gid = lax.broadcasted_iota(jnp.int32, (_G, _BLK), 0)
  onehot = (gid == seg).astype(jnp.float32)

  @pl.when(i == 0)
  def _():
    g_ref[...] = jnp.zeros_like(g_ref)

  g_ref[...] += jnp.dot(onehot, z, preferred_element_type=jnp.float32,
                        precision=lax.Precision.HIGHEST)


def _bn_pool_call(y, s1, s2, gm, bt, seg3):
  full = pl.BlockSpec((1, _H), lambda i: (0, 0))
  rows = pl.BlockSpec((_BLK, _H), lambda i: (i, 0))
  return pl.pallas_call(
      _bn_pool_body,
      grid=(_NBLK,),
      in_specs=[
          rows, full, full, full, full,
          pl.BlockSpec((1, 1, _BLK), lambda i: (i, 0, 0)),
      ],
      out_specs=[rows, pl.BlockSpec((_G, _H), lambda i: (0, 0))],
      out_shape=[
          jax.ShapeDtypeStruct((_N, _H), jnp.float32),
          jax.ShapeDtypeStruct((_G, _H), jnp.float32),
      ],
  )(y, s1, s2, gm, bt, seg3)


def kernel(x, edge_index, batch, W0a, b0a, W0b, b0b, gamma0, beta0,
           W1a, b1a, W1b, b1b, gamma1, beta1,
           W2a, b2a, W2b, b2b, gamma2, beta2):
  src = edge_index[0]
  dst = edge_index[1]
  npad_e = _EPAD - _E
  srcp = jnp.concatenate([src, jnp.zeros((npad_e,), jnp.int32)])
  pad_dst = _N + jnp.arange(npad_e, dtype=jnp.int32) % (_NPAD - _N)
  dstp = jnp.concatenate([dst, pad_dst]).reshape(_NW, _NCHUNK, _CHUNK)
  zero = jnp.zeros((_NPAD, _H), jnp.float32)
  seg3 = batch.reshape(_NBLK, 1, _BLK)
  params = [
      (W0a, b0a, W0b, b0b, gamma0, beta0),
      (W1a, b1a, W1b, b1b, gamma1, beta1),
      (W2a, b2a, W2b, b2b, gamma2, beta2),
  ]
  z = x
  zs = []
  gs = []
  for wa, ba, wb, bb, gm, bt in params:
    acc = _sc_aggregate(z, srcp, dstp, zero)
    y, s1, s2 = _mlp_call(z, acc[0, :_N], acc[1, :_N], wa, ba.reshape(1, _H),
                          wb, bb.reshape(1, _H))
    z, g = _bn_pool_call(y, s1, s2, gm.reshape(1, _H), bt.reshape(1, _H),
                         seg3)
    zs.append(z)
    gs.append(g)
  return jnp.concatenate(zs, axis=1), jnp.concatenate(gs, axis=1)


# speedup vs baseline: 1.0165x; 1.0165x over previous
import functools

import jax
import jax.numpy as jnp
from jax import lax
from jax.experimental import pallas as pl
from jax.experimental.pallas import tpu as pltpu
from jax.experimental.pallas import tpu_sc as plsc

_N = 10000
_E = 320000
_H = 128
_G = 64

_NC = 2
_NS = 16
_NW = _NC * _NS
_CHUNK = 128
_NCHUNK = 80
_EPT = _NCHUNK * _CHUNK
_EPAD = _NW * _EPT
_NBUF = 2
_NPAD = 10112
_RPT = _NPAD // _NS

_BLK = 1000
_NBLK = _N // _BLK


def _sc_aggregate(z, srcp, dstp, zero):
  mesh = plsc.VectorSubcoreMesh(
      core_axis_name="c", subcore_axis_name="s", num_cores=_NC,
      num_subcores=_NS)

  @functools.partial(
      pl.kernel,
      mesh=mesh,
      out_type=jax.ShapeDtypeStruct((_NC, _NPAD, _H), jnp.float32),
      scratch_types=[
          pltpu.VMEM_SHARED((_NPAD, _H), jnp.float32),
          pltpu.VMEM((_NCHUNK, _CHUNK), jnp.int32),
      ] + [pltpu.VMEM((_CHUNK,), jnp.int32) for _ in range(_NBUF)]
        + [pltpu.VMEM((_CHUNK, _H), jnp.float32) for _ in range(_NBUF)]
        + [pltpu.SemaphoreType.DMA for _ in range(3 * _NBUF + 1)],
  )
  def agg_kernel(z_hbm, src_hbm, dst_hbm, zero_hbm, out_hbm,
                 acc, didx, *rest):
    sidx = rest[:_NBUF]
    rows = rest[_NBUF:2 * _NBUF]
    gsems = rest[2 * _NBUF:3 * _NBUF]
    ssems = rest[3 * _NBUF:4 * _NBUF]
    isems = rest[4 * _NBUF:5 * _NBUF]
    zsem = rest[5 * _NBUF]
    cid = lax.axis_index("c")
    sid = lax.axis_index("s")
    wid = sid * _NC + cid
    row0 = sid * _RPT
    pltpu.async_copy(zero_hbm.at[pl.ds(row0, _RPT)],
                     acc.at[pl.ds(row0, _RPT)], zsem)
    ebase = wid * _EPT
    pltpu.async_copy(dst_hbm.at[wid], didx, isems[0])
    for b in range(_NBUF):
      pltpu.async_copy(src_hbm.at[pl.ds(ebase + b * _CHUNK, _CHUNK)],
                       sidx[b], isems[b])
    pltpu.make_async_copy(dst_hbm.at[wid], didx, isems[0]).wait()
    for b in range(_NBUF):
      pltpu.make_async_copy(src_hbm.at[pl.ds(ebase + b * _CHUNK, _CHUNK)],
                            sidx[b], isems[b]).wait()
    pltpu.make_async_copy(zero_hbm.at[pl.ds(row0, _RPT)],
                          acc.at[pl.ds(row0, _RPT)], zsem).wait()
    plsc.subcore_barrier()

    for b in range(_NBUF):
      pltpu.async_copy(z_hbm.at[sidx[b]], rows[b], gsems[b])

    @pl.loop(0, _NCHUNK - _NBUF, step=_NBUF)
    def _(cc):
      for b in range(_NBUF):
        c = cc + b
        pltpu.make_async_copy(z_hbm.at[sidx[b]], rows[b], gsems[b]).wait()
        pltpu.async_copy(
            src_hbm.at[pl.ds(ebase + (c + _NBUF) * _CHUNK, _CHUNK)],
            sidx[b], isems[b])
        pltpu.sync_copy(rows[b], acc.at[didx.at[c]], add=True)
        pltpu.make_async_copy(
            src_hbm.at[pl.ds(ebase + (c + _NBUF) * _CHUNK, _CHUNK)],
            sidx[b], isems[b]).wait()
        pltpu.async_copy(z_hbm.at[sidx[b]], rows[b], gsems[b])

    for b in range(_NBUF):
      c = _NCHUNK - _NBUF + b
      pltpu.make_async_copy(z_hbm.at[sidx[b]], rows[b], gsems[b]).wait()
      pltpu.sync_copy(rows[b], acc.at[didx.at[c]], add=True)

    plsc.subcore_barrier()
    pltpu.sync_copy(acc.at[pl.ds(row0, _RPT)],
                    out_hbm.at[cid, pl.ds(row0, _RPT)])

  return agg_kernel(z, srcp, dstp, zero)


def _mlp_body(z_ref, a0_ref, a1_ref, wa_ref, ba_ref, wb_ref, bb_ref,
              y_ref, s1_ref, s2_ref):
  i = pl.program_id(0)
  h = z_ref[...] + a0_ref[...] + a1_ref[...]
  u = jnp.maximum(
      jnp.dot(h, wa_ref[...], preferred_element_type=jnp.float32)
      + ba_ref[...], 0.0)
  y = jnp.maximum(
      jnp.dot(u, wb_ref[...], preferred_element_type=jnp.float32)
      + bb_ref[...], 0.0)
  y_ref[...] = y

  @pl.when(i == 0)
  def _():
    s1_ref[...] = jnp.zeros_like(s1_ref)
    s2_ref[...] = jnp.zeros_like(s2_ref)

  s1_ref[...] += jnp.sum(y, axis=0, keepdims=True)
  s2_ref[...] += jnp.sum(y * y, axis=0, keepdims=True)


def _mlp_call(z, a0, a1, wa, ba, wb, bb):
  full = pl.BlockSpec((1, _H), lambda i: (0, 0))
  wfull = pl.BlockSpec((_H, _H), lambda i: (0, 0))
  rows = pl.BlockSpec((_BLK, _H), lambda i: (i, 0))
  return pl.pallas_call(
      _mlp_body,
      grid=(_NBLK,),
      in_specs=[rows, rows, rows, wfull, full, wfull, full],
      out_specs=[rows, full, full],
      out_shape=[
          jax.ShapeDtypeStruct((_N, _H), jnp.float32),
          jax.ShapeDtypeStruct((1, _H), jnp.float32),
          jax.ShapeDtypeStruct((1, _H), jnp.float32),
      ],
  )(z, a0, a1, wa, ba, wb, bb)


def _bn_pool_body(y_ref, s1_ref, s2_ref, gm_ref, bt_ref, seg_ref,
                  z_ref, g_ref):
  i = pl.program_id(0)
  mu = s1_ref[...] * (1.0 / _N)
  var = s2_ref[...] * (1.0 / _N) - mu * mu
  a = gm_ref[...] / jnp.sqrt(var + 1e-5)
  b = bt_ref[...] - mu * a
  z = y_ref[...] * a + b
  z_ref[...] = z

  seg = seg_ref[0]
  gid = lax.broadcasted_iota(jnp.int32, (_G, _BLK), 0)
  onehot = (gid == seg).astype(jnp.float32)

  @pl.when(i == 0)
  def _():
    g_ref[...] = jnp.zeros_like(g_ref)

  g_ref[...] += jnp.dot(onehot, z, preferred_element_type=jnp.float32,
                        precision=lax.Precision.HIGHEST)


def _bn_pool_call(y, s1, s2, gm, bt, seg3):
  full = pl.BlockSpec((1, _H), lambda i: (0, 0))
  rows = pl.BlockSpec((_BLK, _H), lambda i: (i, 0))
  return pl.pallas_call(
      _bn_pool_body,
      grid=(_NBLK,),
      in_specs=[
          rows, full, full, full, full,
          pl.BlockSpec((1, 1, _BLK), lambda i: (i, 0, 0)),
      ],
      out_specs=[rows, pl.BlockSpec((_G, _H), lambda i: (0, 0))],
      out_shape=[
          jax.ShapeDtypeStruct((_N, _H), jnp.float32),
          jax.ShapeDtypeStruct((_G, _H), jnp.float32),
      ],
  )(y, s1, s2, gm, bt, seg3)


def kernel(x, edge_index, batch, W0a, b0a, W0b, b0b, gamma0, beta0,
           W1a, b1a, W1b, b1b, gamma1, beta1,
           W2a, b2a, W2b, b2b, gamma2, beta2):
  src = edge_index[0]
  dst = edge_index[1]
  npad_e = _EPAD - _E
  srcp = jnp.concatenate([src, jnp.zeros((npad_e,), jnp.int32)])
  pad_dst = _N + jnp.arange(npad_e, dtype=jnp.int32) % (_NPAD - _N)
  dstp = jnp.concatenate([dst, pad_dst]).reshape(_NW, _NCHUNK, _CHUNK)
  zero = jnp.zeros((_NPAD, _H), jnp.float32)
  seg3 = batch.reshape(_NBLK, 1, _BLK)
  params = [
      (W0a, b0a, W0b, b0b, gamma0, beta0),
      (W1a, b1a, W1b, b1b, gamma1, beta1),
      (W2a, b2a, W2b, b2b, gamma2, beta2),
  ]
  z = x
  zs = []
  gs = []
  for wa, ba, wb, bb, gm, bt in params:
    acc = _sc_aggregate(z, srcp, dstp, zero)
    y, s1, s2 = _mlp_call(z, acc[0, :_N], acc[1, :_N], wa, ba.reshape(1, _H),
                          wb, bb.reshape(1, _H))
    z, g = _bn_pool_call(y, s1, s2, gm.reshape(1, _H), bt.reshape(1, _H),
                         seg3)
    zs.append(z)
    gs.append(g)
  return jnp.concatenate(zs, axis=1), jnp.concatenate(gs, axis=1)


# speedup vs baseline: 2.0429x; 2.0099x over previous
import functools

import jax
import jax.numpy as jnp
from jax import lax
from jax.experimental import pallas as pl
from jax.experimental.pallas import tpu as pltpu
from jax.experimental.pallas import tpu_sc as plsc

_N = 10000
_E = 320000
_H = 128
_G = 64

_NC = 2
_NS = 16
_NW = _NC * _NS
_CHUNK = 80
_NCHUNK = 126
_EPT = _NCHUNK * _CHUNK
_EPAD = _NW * _EPT
_NBUF = 2
_NPAD = 10112
_RPT = _NPAD // _NS

_BLK = 1000
_NBLK = _N // _BLK


def _sc_aggregate(z, srcp, dstp, zero):
  mesh = plsc.VectorSubcoreMesh(
      core_axis_name="c", subcore_axis_name="s", num_cores=_NC,
      num_subcores=_NS)

  @functools.partial(
      pl.kernel,
      mesh=mesh,
      out_type=jax.ShapeDtypeStruct((_NC, _NPAD, _H), jnp.float32),
      scratch_types=[
          pltpu.VMEM_SHARED((_NPAD, _H), jnp.float32),
          pltpu.VMEM((_NCHUNK, _CHUNK), jnp.int32),
      ] + [pltpu.VMEM((_CHUNK,), jnp.int32) for _ in range(_NBUF)]
        + [pltpu.VMEM((_CHUNK, _H), jnp.float32) for _ in range(_NBUF)]
        + [pltpu.SemaphoreType.DMA for _ in range(3 * _NBUF + 1)],
  )
  def agg_kernel(z_hbm, src_hbm, dst_hbm, zero_hbm, out_hbm,
                 acc, didx, *rest):
    sidx = rest[:_NBUF]
    rows = rest[_NBUF:2 * _NBUF]
    gsems = rest[2 * _NBUF:3 * _NBUF]
    ssems = rest[3 * _NBUF:4 * _NBUF]
    isems = rest[4 * _NBUF:5 * _NBUF]
    zsem = rest[5 * _NBUF]
    cid = lax.axis_index("c")
    sid = lax.axis_index("s")
    wid = sid * _NC + cid
    row0 = sid * _RPT
    pltpu.async_copy(zero_hbm.at[pl.ds(row0, _RPT)],
                     acc.at[pl.ds(row0, _RPT)], zsem)
    ebase = wid * _EPT
    pltpu.async_copy(dst_hbm.at[wid], didx, isems[0])
    for b in range(_NBUF):
      pltpu.async_copy(src_hbm.at[pl.ds(ebase + b * _CHUNK, _CHUNK)],
                       sidx[b], isems[b])
    pltpu.make_async_copy(dst_hbm.at[wid], didx, isems[0]).wait()
    for b in range(_NBUF):
      pltpu.make_async_copy(src_hbm.at[pl.ds(ebase + b * _CHUNK, _CHUNK)],
                            sidx[b], isems[b]).wait()
    pltpu.make_async_copy(zero_hbm.at[pl.ds(row0, _RPT)],
                          acc.at[pl.ds(row0, _RPT)], zsem).wait()
    plsc.subcore_barrier()

    for b in range(_NBUF):
      pltpu.async_copy(z_hbm.at[sidx[b]], rows[b], gsems[b])

    @pl.loop(0, _NCHUNK - _NBUF, step=_NBUF)
    def _(cc):
      for b in range(_NBUF):
        c = cc + b
        pltpu.make_async_copy(z_hbm.at[sidx[b]], rows[b], gsems[b]).wait()
        pltpu.async_copy(
            src_hbm.at[pl.ds(ebase + (c + _NBUF) * _CHUNK, _CHUNK)],
            sidx[b], isems[b])
        pltpu.sync_copy(rows[b], acc.at[didx.at[c]], add=True)
        pltpu.make_async_copy(
            src_hbm.at[pl.ds(ebase + (c + _NBUF) * _CHUNK, _CHUNK)],
            sidx[b], isems[b]).wait()
        pltpu.async_copy(z_hbm.at[sidx[b]], rows[b], gsems[b])

    for b in range(_NBUF):
      c = _NCHUNK - _NBUF + b
      pltpu.make_async_copy(z_hbm.at[sidx[b]], rows[b], gsems[b]).wait()
      pltpu.sync_copy(rows[b], acc.at[didx.at[c]], add=True)

    plsc.subcore_barrier()
    pltpu.sync_copy(acc.at[pl.ds(row0, _RPT)],
                    out_hbm.at[cid, pl.ds(row0, _RPT)])

  return agg_kernel(z, srcp, dstp, zero)


def _mlp_body(z_ref, a0_ref, a1_ref, wa_ref, ba_ref, wb_ref, bb_ref,
              y_ref, s1_ref, s2_ref):
  i = pl.program_id(0)
  h = z_ref[...] + a0_ref[...] + a1_ref[...]
  u = jnp.maximum(
      jnp.dot(h, wa_ref[...], preferred_element_type=jnp.float32)
      + ba_ref[...], 0.0)
  y = jnp.maximum(
      jnp.dot(u, wb_ref[...], preferred_element_type=jnp.float32)
      + bb_ref[...], 0.0)
  y_ref[...] = y

  @pl.when(i == 0)
  def _():
    s1_ref[...] = jnp.zeros_like(s1_ref)
    s2_ref[...] = jnp.zeros_like(s2_ref)

  s1_ref[...] += jnp.sum(y, axis=0, keepdims=True)
  s2_ref[...] += jnp.sum(y * y, axis=0, keepdims=True)


def _mlp_call(z, a0, a1, wa, ba, wb, bb):
  full = pl.BlockSpec((1, _H), lambda i: (0, 0))
  wfull = pl.BlockSpec((_H, _H), lambda i: (0, 0))
  rows = pl.BlockSpec((_BLK, _H), lambda i: (i, 0))
  return pl.pallas_call(
      _mlp_body,
      grid=(_NBLK,),
      in_specs=[rows, rows, rows, wfull, full, wfull, full],
      out_specs=[rows, full, full],
      out_shape=[
          jax.ShapeDtypeStruct((_N, _H), jnp.float32),
          jax.ShapeDtypeStruct((1, _H), jnp.float32),
          jax.ShapeDtypeStruct((1, _H), jnp.float32),
      ],
  )(z, a0, a1, wa, ba, wb, bb)


def _bn_pool_body(y_ref, s1_ref, s2_ref, gm_ref, bt_ref, seg_ref,
                  z_ref, g_ref):
  i = pl.program_id(0)
  mu = s1_ref[...] * (1.0 / _N)
  var = s2_ref[...] * (1.0 / _N) - mu * mu
  a = gm_ref[...] / jnp.sqrt(var + 1e-5)
  b = bt_ref[...] - mu * a
  z = y_ref[...] * a + b
  z_ref[...] = z

  seg = seg_ref[0]
  gid = lax.broadcasted_iota(jnp.int32, (_G, _BLK), 0)
  onehot = (gid == seg).astype(jnp.float32)

  @pl.when(i == 0)
  def _():
    g_ref[...] = jnp.zeros_like(g_ref)

  g_ref[...] += jnp.dot(onehot, z, preferred_element_type=jnp.float32,
                        precision=lax.Precision.HIGHEST)


def _bn_pool_call(y, s1, s2, gm, bt, seg3):
  full = pl.BlockSpec((1, _H), lambda i: (0, 0))
  rows = pl.BlockSpec((_BLK, _H), lambda i: (i, 0))
  return pl.pallas_call(
      _bn_pool_body,
      grid=(_NBLK,),
      in_specs=[
          rows, full, full, full, full,
          pl.BlockSpec((1, 1, _BLK), lambda i: (i, 0, 0)),
      ],
      out_specs=[rows, pl.BlockSpec((_G, _H), lambda i: (0, 0))],
      out_shape=[
          jax.ShapeDtypeStruct((_N, _H), jnp.float32),
          jax.ShapeDtypeStruct((_G, _H), jnp.float32),
      ],
  )(y, s1, s2, gm, bt, seg3)


def kernel(x, edge_index, batch, W0a, b0a, W0b, b0b, gamma0, beta0,
           W1a, b1a, W1b, b1b, gamma1, beta1,
           W2a, b2a, W2b, b2b, gamma2, beta2):
  src = edge_index[0]
  dst = edge_index[1]
  npad_e = _EPAD - _E
  srcp = jnp.concatenate([src, jnp.zeros((npad_e,), jnp.int32)])
  pad_dst = _N + jnp.arange(npad_e, dtype=jnp.int32) % (_NPAD - _N)
  dstp = jnp.concatenate([dst, pad_dst]).reshape(_NW, _NCHUNK, _CHUNK)
  zero = jnp.zeros((_NPAD, _H), jnp.float32)
  seg3 = batch.reshape(_NBLK, 1, _BLK)
  params = [
      (W0a, b0a, W0b, b0b, gamma0, beta0),
      (W1a, b1a, W1b, b1b, gamma1, beta1),
      (W2a, b2a, W2b, b2b, gamma2, beta2),
  ]
  z = x
  zs = []
  gs = []
  for wa, ba, wb, bb, gm, bt in params:
    acc = _sc_aggregate(z, srcp, dstp, zero)
    y, s1, s2 = _mlp_call(z, acc[0, :_N], acc[1, :_N], wa, ba.reshape(1, _H),
                          wb, bb.reshape(1, _H))
    z, g = _bn_pool_call(y, s1, s2, gm.reshape(1, _H), bt.reshape(1, _H),
                         seg3)
    zs.append(z)
    gs.append(g)
  return jnp.concatenate(zs, axis=1), jnp.concatenate(gs, axis=1)


# speedup vs baseline: 2.1310x; 1.0431x over previous
import functools

import jax
import jax.numpy as jnp
from jax import lax
from jax.experimental import pallas as pl
from jax.experimental.pallas import tpu as pltpu
from jax.experimental.pallas import tpu_sc as plsc

_N = 10000
_E = 320000
_H = 128
_G = 64

_NC = 2
_NS = 16
_NW = _NC * _NS
_CHUNK = 80
_NCHUNK = 126
_EPT = _NCHUNK * _CHUNK
_EPAD = _NW * _EPT
_NBUF = 3
_NPAD = 10112
_RPT = _NPAD // _NS

_BLK = 1000
_NBLK = _N // _BLK


def _sc_aggregate(z, srcp, dstp, zero):
  mesh = plsc.VectorSubcoreMesh(
      core_axis_name="c", subcore_axis_name="s", num_cores=_NC,
      num_subcores=_NS)

  @functools.partial(
      pl.kernel,
      mesh=mesh,
      out_type=jax.ShapeDtypeStruct((_NC, _NPAD, _H), jnp.float32),
      scratch_types=[
          pltpu.VMEM_SHARED((_NPAD, _H), jnp.float32),
          pltpu.VMEM((_NCHUNK, _CHUNK), jnp.int32),
      ] + [pltpu.VMEM((_CHUNK,), jnp.int32) for _ in range(_NBUF)]
        + [pltpu.VMEM((_CHUNK, _H), jnp.float32) for _ in range(_NBUF)]
        + [pltpu.SemaphoreType.DMA for _ in range(3 * _NBUF + 1)],
  )
  def agg_kernel(z_hbm, src_hbm, dst_hbm, zero_hbm, out_hbm,
                 acc, didx, *rest):
    sidx = rest[:_NBUF]
    rows = rest[_NBUF:2 * _NBUF]
    gsems = rest[2 * _NBUF:3 * _NBUF]
    ssems = rest[3 * _NBUF:4 * _NBUF]
    isems = rest[4 * _NBUF:5 * _NBUF]
    zsem = rest[5 * _NBUF]
    cid = lax.axis_index("c")
    sid = lax.axis_index("s")
    wid = sid * _NC + cid
    row0 = sid * _RPT
    pltpu.async_copy(zero_hbm.at[pl.ds(row0, _RPT)],
                     acc.at[pl.ds(row0, _RPT)], zsem)
    ebase = wid * _EPT
    pltpu.async_copy(dst_hbm.at[wid], didx, isems[0])
    for b in range(_NBUF):
      pltpu.async_copy(src_hbm.at[pl.ds(ebase + b * _CHUNK, _CHUNK)],
                       sidx[b], isems[b])
    pltpu.make_async_copy(dst_hbm.at[wid], didx, isems[0]).wait()
    for b in range(_NBUF):
      pltpu.make_async_copy(src_hbm.at[pl.ds(ebase + b * _CHUNK, _CHUNK)],
                            sidx[b], isems[b]).wait()
    pltpu.make_async_copy(zero_hbm.at[pl.ds(row0, _RPT)],
                          acc.at[pl.ds(row0, _RPT)], zsem).wait()
    plsc.subcore_barrier()

    for b in range(_NBUF):
      pltpu.async_copy(z_hbm.at[sidx[b]], rows[b], gsems[b])

    @pl.loop(0, _NCHUNK - _NBUF, step=_NBUF)
    def _(cc):
      for b in range(_NBUF):
        c = cc + b
        pltpu.make_async_copy(z_hbm.at[sidx[b]], rows[b], gsems[b]).wait()
        pltpu.async_copy(
            src_hbm.at[pl.ds(ebase + (c + _NBUF) * _CHUNK, _CHUNK)],
            sidx[b], isems[b])
        pltpu.sync_copy(rows[b], acc.at[didx.at[c]], add=True)
        pltpu.make_async_copy(
            src_hbm.at[pl.ds(ebase + (c + _NBUF) * _CHUNK, _CHUNK)],
            sidx[b], isems[b]).wait()
        pltpu.async_copy(z_hbm.at[sidx[b]], rows[b], gsems[b])

    for b in range(_NBUF):
      c = _NCHUNK - _NBUF + b
      pltpu.make_async_copy(z_hbm.at[sidx[b]], rows[b], gsems[b]).wait()
      pltpu.sync_copy(rows[b], acc.at[didx.at[c]], add=True)

    plsc.subcore_barrier()
    pltpu.sync_copy(acc.at[pl.ds(row0, _RPT)],
                    out_hbm.at[cid, pl.ds(row0, _RPT)])

  return agg_kernel(z, srcp, dstp, zero)


def _mlp_body(z_ref, a0_ref, a1_ref, wa_ref, ba_ref, wb_ref, bb_ref,
              y_ref, s1_ref, s2_ref):
  i = pl.program_id(0)
  h = z_ref[...] + a0_ref[...] + a1_ref[...]
  u = jnp.maximum(
      jnp.dot(h, wa_ref[...], preferred_element_type=jnp.float32)
      + ba_ref[...], 0.0)
  y = jnp.maximum(
      jnp.dot(u, wb_ref[...], preferred_element_type=jnp.float32)
      + bb_ref[...], 0.0)
  y_ref[...] = y

  @pl.when(i == 0)
  def _():
    s1_ref[...] = jnp.zeros_like(s1_ref)
    s2_ref[...] = jnp.zeros_like(s2_ref)

  s1_ref[...] += jnp.sum(y, axis=0, keepdims=True)
  s2_ref[...] += jnp.sum(y * y, axis=0, keepdims=True)


def _mlp_call(z, a0, a1, wa, ba, wb, bb):
  full = pl.BlockSpec((1, _H), lambda i: (0, 0))
  wfull = pl.BlockSpec((_H, _H), lambda i: (0, 0))
  rows = pl.BlockSpec((_BLK, _H), lambda i: (i, 0))
  return pl.pallas_call(
      _mlp_body,
      grid=(_NBLK,),
      in_specs=[rows, rows, rows, wfull, full, wfull, full],
      out_specs=[rows, full, full],
      out_shape=[
          jax.ShapeDtypeStruct((_N, _H), jnp.float32),
          jax.ShapeDtypeStruct((1, _H), jnp.float32),
          jax.ShapeDtypeStruct((1, _H), jnp.float32),
      ],
  )(z, a0, a1, wa, ba, wb, bb)


def _bn_pool_body(y_ref, s1_ref, s2_ref, gm_ref, bt_ref, seg_ref,
                  z_ref, g_ref):
  i = pl.program_id(0)
  mu = s1_ref[...] * (1.0 / _N)
  var = s2_ref[...] * (1.0 / _N) - mu * mu
  a = gm_ref[...] / jnp.sqrt(var + 1e-5)
  b = bt_ref[...] - mu * a
  z = y_ref[...] * a + b
  z_ref[...] = z

  seg = seg_ref[0]
  gid = lax.broadcasted_iota(jnp.int32, (_G, _BLK), 0)
  onehot = (gid == seg).astype(jnp.float32)

  @pl.when(i == 0)
  def _():
    g_ref[...] = jnp.zeros_like(g_ref)

  g_ref[...] += jnp.dot(onehot, z, preferred_element_type=jnp.float32,
                        precision=lax.Precision.HIGHEST)


def _bn_pool_call(y, s1, s2, gm, bt, seg3):
  full = pl.BlockSpec((1, _H), lambda i: (0, 0))
  rows = pl.BlockSpec((_BLK, _H), lambda i: (i, 0))
  return pl.pallas_call(
      _bn_pool_body,
      grid=(_NBLK,),
      in_specs=[
          rows, full, full, full, full,
          pl.BlockSpec((1, 1, _BLK), lambda i: (i, 0, 0)),
      ],
      out_specs=[rows, pl.BlockSpec((_G, _H), lambda i: (0, 0))],
      out_shape=[
          jax.ShapeDtypeStruct((_N, _H), jnp.float32),
          jax.ShapeDtypeStruct((_G, _H), jnp.float32),
      ],
  )(y, s1, s2, gm, bt, seg3)


def kernel(x, edge_index, batch, W0a, b0a, W0b, b0b, gamma0, beta0,
           W1a, b1a, W1b, b1b, gamma1, beta1,
           W2a, b2a, W2b, b2b, gamma2, beta2):
  src = edge_index[0]
  dst = edge_index[1]
  npad_e = _EPAD - _E
  srcp = jnp.concatenate([src, jnp.zeros((npad_e,), jnp.int32)])
  pad_dst = _N + jnp.arange(npad_e, dtype=jnp.int32) % (_NPAD - _N)
  dstp = jnp.concatenate([dst, pad_dst]).reshape(_NW, _NCHUNK, _CHUNK)
  zero = jnp.zeros((_NPAD, _H), jnp.float32)
  seg3 = batch.reshape(_NBLK, 1, _BLK)
  params = [
      (W0a, b0a, W0b, b0b, gamma0, beta0),
      (W1a, b1a, W1b, b1b, gamma1, beta1),
      (W2a, b2a, W2b, b2b, gamma2, beta2),
  ]
  z = x
  zs = []
  gs = []
  for wa, ba, wb, bb, gm, bt in params:
    acc = _sc_aggregate(z, srcp, dstp, zero)
    y, s1, s2 = _mlp_call(z, acc[0, :_N], acc[1, :_N], wa, ba.reshape(1, _H),
                          wb, bb.reshape(1, _H))
    z, g = _bn_pool_call(y, s1, s2, gm.reshape(1, _H), bt.reshape(1, _H),
                         seg3)
    zs.append(z)
    gs.append(g)
  return jnp.concatenate(zs, axis=1), jnp.concatenate(gs, axis=1)
